# baseline (speedup 1.0000x reference)
import functools

import jax
import jax.numpy as jnp
import numpy as np
from jax import lax
from jax.experimental import pallas as pl
from jax.experimental.pallas import tpu as pltpu
from jax.experimental.pallas import tpu_sc as plsc

N = 10000
E = 160000
C = 10
DL = 16
RL = 6
K_PROP = 10

NT = 16
CW = 128
CH = 80
EPT = CH * CW
NPT = N // NT
NACC = N + 16
NS = 8
GA = 4

_LOG_SCALE = 0.5 * DL * np.log(4.0 * np.pi)
_LOG2PI = np.log(2.0 * np.pi)


def _encoder_body(y_ref, x_ref, W1_ref, b1_ref, W2T_ref, b2T_ref, z0T_ref,
                  a_ref, b_ref, out_ref, logpc_scr):
    blk = x_ref.shape[0]

    @pl.when(pl.program_id(0) == 0)
    def _():
        y = y_ref[...]
        col = lax.broadcasted_iota(jnp.int32, (1, 128), 1)
        row = jnp.zeros((1, 128), jnp.float32)
        for c in range(C):
            cnt = jnp.sum(jnp.where(y == c, 1.0, 0.0))
            row = row + jnp.where(col == c, cnt, 0.0)
        logpc_scr[...] = jnp.where(col < C,
                                   jnp.log(row * (1.0 / float(N))), 0.0)

    h = jnp.maximum(
        jnp.dot(x_ref[...], W1_ref[...], preferred_element_type=jnp.float32)
        + b1_ref[...], 0.0)
    zT = lax.dot_general(W2T_ref[...], h, (((1,), (1,)), ((), ())),
                         preferred_element_type=jnp.float32) + b2T_ref[...]
    a_sp = jax.nn.softplus(a_ref[...])
    b_sp = jax.nn.softplus(b_ref[...])
    rows = []
    for c in range(C):
        zc = zT
        ldet = jnp.zeros((1, blk), jnp.float32)
        for l in range(RL):
            j = c * RL + l
            alpha = a_sp[:, j:j + 1]
            beta = b_sp[:, j:j + 1] - alpha
            z0 = z0T_ref[:, j:j + 1]
            diff = zc - z0
            r = jnp.sqrt(jnp.sum(diff * diff, axis=0, keepdims=True) + 1e-12)
            hh = 1.0 / (alpha + r)
            bh = beta * hh
            zc = zc + bh * diff
            ldet = ldet + (DL - 1) * jnp.log1p(bh) \
                + jnp.log1p(bh - beta * hh * hh * r)
        logq = -0.5 * (DL * _LOG2PI
                       + jnp.sum(zc * zc, axis=0, keepdims=True)) + ldet
        lp = logpc_scr[:, c:c + 1]
        rows.append(jnp.exp(jnp.clip(logq + lp + _LOG_SCALE, -30.0, 30.0)))
    out_ref[...] = jnp.concatenate(
        rows + [jnp.zeros((DL - C, blk), jnp.float32)], axis=0)


@functools.cache
def _make_appnp_sc():
    mesh = plsc.VectorSubcoreMesh(core_axis_name="c", subcore_axis_name="s",
                                  num_cores=1, num_subcores=NT)
    return pl.kernel(
        _appnp_body,
        out_type=[jax.ShapeDtypeStruct((N, DL), jnp.float32),
                  jax.ShapeDtypeStruct((NACC, DL), jnp.float32)],
        mesh=mesh,
        scratch_types=[
            pltpu.VMEM((CH, CW), jnp.int32),
            pltpu.VMEM((CH, CW), jnp.int32),
            pltpu.VMEM((NS, CW, DL), jnp.float32),
            pltpu.VMEM((NPT, DL), jnp.float32),
            pltpu.VMEM((NPT, DL), jnp.float32),
            pltpu.VMEM((NPT, DL), jnp.float32),
            pltpu.VMEM((NPT, DL), jnp.float32),
            pltpu.VMEM((NPT, DL), jnp.float32),
            pltpu.VMEM((NPT, DL), jnp.float32),
            pltpu.VMEM((NPT, DL), jnp.float32),
            pltpu.VMEM((CW, DL), jnp.float32),
            pltpu.VMEM_SHARED((NACC, DL), jnp.float32),
            pltpu.SemaphoreType.DMA((NS,)),
            pltpu.SemaphoreType.DMA((NS,)),
        ],
        compiler_params=pltpu.CompilerParams(use_tc_tiling_on_sc=False,
                                             needs_layout_passes=False),
    )


def _appnp_body(src_hbm, dst_hbm, x0_hbm, soft_hbm, g_hbm,
                idx_s, idx_d, gbuf, accv, gv, u9v, v0v, dinvv, x0v, zv, onesv,
                acc_s, gsem, ssem):
    wid = lax.axis_index("s")
    base = wid * NPT

    pltpu.sync_copy(src_hbm.at[wid], idx_s)
    pltpu.sync_copy(dst_hbm.at[wid], idx_d)
    pltpu.sync_copy(x0_hbm.at[pl.ds(base, NPT)], x0v)

    ones16 = jnp.ones((DL,), jnp.float32)
    zero16 = jnp.zeros((DL,), jnp.float32)

    @pl.loop(0, CW, unroll=8)
    def _(i):
        onesv[i] = ones16

    @pl.loop(0, NPT, unroll=8)
    def _(i):
        zv[i] = zero16

    pltpu.sync_copy(zv, acc_s.at[pl.ds(base, NPT)])

    @pl.when(wid == 0)
    def _():
        pltpu.sync_copy(zv.at[pl.ds(0, 16)], acc_s.at[pl.ds(N, 16)])
        pltpu.sync_copy(zv.at[pl.ds(0, 16)], g_hbm.at[pl.ds(N, 16)])

    plsc.subcore_barrier()

    @pl.loop(0, CH)
    def _(c):
        s = lax.rem(c, NS)

        @pl.when(c >= NS)
        def _():
            pltpu.make_async_copy(onesv, acc_s.at[idx_d.at[0]],
                                  ssem.at[s]).wait()

        pltpu.async_copy(onesv, acc_s.at[idx_d.at[c]], ssem.at[s], add=True)

    for s in range(NS):
        pltpu.make_async_copy(onesv, acc_s.at[idx_d.at[0]], ssem.at[s]).wait()

    plsc.subcore_barrier()

    pltpu.sync_copy(acc_s.at[pl.ds(base, NPT)], accv)
    magic = jnp.full((DL,), 0x5F3759DF, jnp.int32)

    @pl.loop(0, NPT, unroll=4)
    def _(i):
        deg = accv[i] + 1.0
        yi = lax.bitcast_convert_type(
            magic - lax.shift_right_arithmetic(
                lax.bitcast_convert_type(deg, jnp.int32), 1), jnp.float32)
        yi = yi * (1.5 - 0.5 * deg * yi * yi)
        yi = yi * (1.5 - 0.5 * deg * yi * yi)
        yi = yi * (1.5 - 0.5 * deg * yi * yi)
        g0 = yi * x0v[i]
        gv[i] = g0
        u9v[i] = 0.9 * (yi * yi)
        v0v[i] = 0.1 * g0
        dinvv[i] = yi

    pltpu.sync_copy(zv, acc_s.at[pl.ds(base, NPT)])
    pltpu.sync_copy(gv, g_hbm.at[pl.ds(base, NPT)])
    plsc.subcore_barrier()

    def edge_pass():
        for s in range(GA):
            pltpu.async_copy(g_hbm.at[idx_s.at[s]], gbuf.at[s], gsem.at[s])

        @pl.loop(0, CH)
        def _(cc):
            nxt = cc + GA
            s_n = lax.rem(nxt, NS)

            @pl.when(nxt < CH)
            def _():
                @pl.when(nxt >= NS)
                def _():
                    pltpu.make_async_copy(gbuf.at[s_n],
                                          acc_s.at[idx_d.at[nxt - NS]],
                                          ssem.at[s_n]).wait()

                pltpu.async_copy(g_hbm.at[idx_s.at[nxt]], gbuf.at[s_n],
                                 gsem.at[s_n])

            s = lax.rem(cc, NS)
            pltpu.make_async_copy(g_hbm.at[idx_s.at[cc]], gbuf.at[s],
                                  gsem.at[s]).wait()
            pltpu.async_copy(gbuf.at[s], acc_s.at[idx_d.at[cc]], ssem.at[s],
                             add=True)

        for s in range(NS):
            pltpu.make_async_copy(gbuf.at[s], acc_s.at[idx_d.at[0]],
                                  ssem.at[s]).wait()

    @pl.loop(0, K_PROP - 1)
    def _(k):
        edge_pass()
        plsc.subcore_barrier()
        pltpu.sync_copy(acc_s.at[pl.ds(base, NPT)], accv)
        pltpu.sync_copy(zv, acc_s.at[pl.ds(base, NPT)])

        @pl.loop(0, NPT, unroll=4)
        def _(i):
            gv[i] = u9v[i] * (accv[i] + gv[i]) + v0v[i]

        pltpu.sync_copy(gv, g_hbm.at[pl.ds(base, NPT)])
        plsc.subcore_barrier()

    edge_pass()
    plsc.subcore_barrier()
    pltpu.sync_copy(acc_s.at[pl.ds(base, NPT)], accv)

    @pl.loop(0, NPT, unroll=4)
    def _(i):
        h = 0.9 * dinvv[i] * (accv[i] + gv[i]) + 0.1 * x0v[i]
        tot = float(C) + jnp.sum(h)
        accv[i] = (1.0 + h) / tot

    pltpu.sync_copy(accv, soft_hbm.at[pl.ds(base, NPT)])


def kernel(x, edge_index, y, W1, b1, W2, b2, flow_z0, flow_alpha_raw,
           flow_beta_raw):
    ypad = jnp.concatenate(
        [y.astype(jnp.int32), jnp.full((80 * 128 - N,), C, jnp.int32)]
    ).reshape(80, 128)

    BLK = 1024
    NENC = 10 * BLK
    xpad = jnp.concatenate([x, jnp.zeros((NENC - N, 256), jnp.float32)], 0)
    z0T = flow_z0.reshape(C * RL, DL).T
    z0T = jnp.concatenate([z0T, jnp.zeros((DL, 64 - C * RL), jnp.float32)], 1)
    ar = jnp.concatenate([flow_alpha_raw.reshape(1, C * RL),
                          jnp.zeros((1, 64 - C * RL), jnp.float32)], 1)
    br = jnp.concatenate([flow_beta_raw.reshape(1, C * RL),
                          jnp.zeros((1, 64 - C * RL), jnp.float32)], 1)
    x0T = pl.pallas_call(
        _encoder_body,
        grid=(NENC // BLK,),
        in_specs=[
            pl.BlockSpec((80, 128), lambda i: (0, 0)),
            pl.BlockSpec((BLK, 256), lambda i: (i, 0)),
            pl.BlockSpec((256, 64), lambda i: (0, 0)),
            pl.BlockSpec((1, 64), lambda i: (0, 0)),
            pl.BlockSpec((DL, 64), lambda i: (0, 0)),
            pl.BlockSpec((DL, 1), lambda i: (0, 0)),
            pl.BlockSpec((DL, 64), lambda i: (0, 0)),
            pl.BlockSpec((1, 64), lambda i: (0, 0)),
            pl.BlockSpec((1, 64), lambda i: (0, 0)),
        ],
        out_specs=pl.BlockSpec((DL, BLK), lambda i: (0, i)),
        out_shape=jax.ShapeDtypeStruct((DL, NENC), jnp.float32),
        scratch_shapes=[pltpu.VMEM((1, 128), jnp.float32)],
    )(ypad, xpad, W1, b1.reshape(1, 64), W2.T, b2.reshape(DL, 1), z0T, ar, br)
    x0 = x0T.T[:N]

    src = edge_index[0].astype(jnp.int32)
    dst = edge_index[1].astype(jnp.int32)
    pad = NT * EPT - E
    src_pad = jnp.concatenate([src, jnp.zeros((pad,), jnp.int32)]
                              ).reshape(NT, CH, CW)
    dst_pad = jnp.concatenate([dst, jnp.full((pad,), N, jnp.int32)]
                              ).reshape(NT, CH, CW)

    soft16, _ = _make_appnp_sc()(src_pad, dst_pad, x0)
    return soft16[:, :C]


# speedup vs baseline: 1.0577x; 1.0577x over previous
import functools

import jax
import jax.numpy as jnp
import numpy as np
from jax import lax
from jax.experimental import pallas as pl
from jax.experimental.pallas import tpu as pltpu
from jax.experimental.pallas import tpu_sc as plsc

N = 10000
E = 160000
C = 10
DL = 16
RL = 6
K_PROP = 10

NT = 16
CW = 128
CH = 80
EPT = CH * CW
NPT = N // NT
NACC = N + 16
NS = 12
GA = 6

_LOG_SCALE = 0.5 * DL * np.log(4.0 * np.pi)
_LOG2PI = np.log(2.0 * np.pi)


def _encoder_body(y_ref, x_ref, W1_ref, b1_ref, W2T_ref, b2T_ref, z0T_ref,
                  a_ref, b_ref, out_ref, logpc_scr):
    blk = x_ref.shape[0]

    @pl.when(pl.program_id(0) == 0)
    def _():
        y = y_ref[...]
        col = lax.broadcasted_iota(jnp.int32, (1, 128), 1)
        row = jnp.zeros((1, 128), jnp.float32)
        for c in range(C):
            cnt = jnp.sum(jnp.where(y == c, 1.0, 0.0))
            row = row + jnp.where(col == c, cnt, 0.0)
        logpc_scr[...] = jnp.where(col < C,
                                   jnp.log(row * (1.0 / float(N))), 0.0)

    h = jnp.maximum(
        jnp.dot(x_ref[...], W1_ref[...], preferred_element_type=jnp.float32)
        + b1_ref[...], 0.0)
    zT = lax.dot_general(W2T_ref[...], h, (((1,), (1,)), ((), ())),
                         preferred_element_type=jnp.float32) + b2T_ref[...]
    a_sp = jax.nn.softplus(a_ref[...])
    b_sp = jax.nn.softplus(b_ref[...])
    rows = []
    for c in range(C):
        zc = zT
        ldet = jnp.zeros((1, blk), jnp.float32)
        for l in range(RL):
            j = c * RL + l
            alpha = a_sp[:, j:j + 1]
            beta = b_sp[:, j:j + 1] - alpha
            z0 = z0T_ref[:, j:j + 1]
            diff = zc - z0
            r = jnp.sqrt(jnp.sum(diff * diff, axis=0, keepdims=True) + 1e-12)
            hh = 1.0 / (alpha + r)
            bh = beta * hh
            zc = zc + bh * diff
            ldet = ldet + (DL - 1) * jnp.log1p(bh) \
                + jnp.log1p(bh - beta * hh * hh * r)
        logq = -0.5 * (DL * _LOG2PI
                       + jnp.sum(zc * zc, axis=0, keepdims=True)) + ldet
        lp = logpc_scr[:, c:c + 1]
        rows.append(jnp.exp(jnp.clip(logq + lp + _LOG_SCALE, -30.0, 30.0)))
    out_ref[...] = jnp.concatenate(
        rows + [jnp.zeros((DL - C, blk), jnp.float32)], axis=0)


@functools.cache
def _make_appnp_sc():
    mesh = plsc.VectorSubcoreMesh(core_axis_name="c", subcore_axis_name="s",
                                  num_cores=1, num_subcores=NT)
    return pl.kernel(
        _appnp_body,
        out_type=[jax.ShapeDtypeStruct((N, DL), jnp.float32),
                  jax.ShapeDtypeStruct((NACC, DL), jnp.float32)],
        mesh=mesh,
        scratch_types=[
            pltpu.VMEM((CH, CW), jnp.int32),
            pltpu.VMEM((CH, CW), jnp.int32),
            pltpu.VMEM((NS, CW, DL), jnp.float32),
            pltpu.VMEM((NPT, DL), jnp.float32),
            pltpu.VMEM((NPT, DL), jnp.float32),
            pltpu.VMEM((NPT, DL), jnp.float32),
            pltpu.VMEM((NPT, DL), jnp.float32),
            pltpu.VMEM((NPT, DL), jnp.float32),
            pltpu.VMEM((NPT, DL), jnp.float32),
            pltpu.VMEM((NPT, DL), jnp.float32),
            pltpu.VMEM((CW, DL), jnp.float32),
            pltpu.VMEM_SHARED((NACC, DL), jnp.float32),
            pltpu.SemaphoreType.DMA((NS,)),
            pltpu.SemaphoreType.DMA((NS,)),
        ],
        compiler_params=pltpu.CompilerParams(use_tc_tiling_on_sc=False,
                                             needs_layout_passes=False),
    )


def _appnp_body(src_hbm, dst_hbm, x0_hbm, soft_hbm, g_hbm,
                idx_s, idx_d, gbuf, accv, gv, u9v, v0v, dinvv, x0v, zv, onesv,
                acc_s, gsem, ssem):
    wid = lax.axis_index("s")
    base = wid * NPT

    pltpu.sync_copy(src_hbm.at[wid], idx_s)
    pltpu.sync_copy(dst_hbm.at[wid], idx_d)
    pltpu.sync_copy(x0_hbm.at[pl.ds(base, NPT)], x0v)

    ones16 = jnp.ones((DL,), jnp.float32)
    zero16 = jnp.zeros((DL,), jnp.float32)

    @pl.loop(0, CW)
    def _(i):
        onesv[i] = ones16

    @pl.loop(0, NPT)
    def _(i):
        zv[i] = zero16

    pltpu.sync_copy(zv, acc_s.at[pl.ds(base, NPT)])

    @pl.when(wid == 0)
    def _():
        pltpu.sync_copy(zv.at[pl.ds(0, 16)], acc_s.at[pl.ds(N, 16)])
        pltpu.sync_copy(zv.at[pl.ds(0, 16)], g_hbm.at[pl.ds(N, 16)])

    plsc.subcore_barrier()

    @pl.loop(0, CH)
    def _(c):
        s = lax.rem(c, NS)

        @pl.when(c >= NS)
        def _():
            pltpu.make_async_copy(onesv, acc_s.at[idx_d.at[0]],
                                  ssem.at[s]).wait()

        pltpu.async_copy(onesv, acc_s.at[idx_d.at[c]], ssem.at[s], add=True)

    for s in range(NS):
        pltpu.make_async_copy(onesv, acc_s.at[idx_d.at[0]], ssem.at[s]).wait()

    plsc.subcore_barrier()

    pltpu.sync_copy(acc_s.at[pl.ds(base, NPT)], accv)
    magic = jnp.full((DL,), 0x5F3759DF, jnp.int32)

    @pl.loop(0, NPT)
    def _(i):
        deg = accv[i] + 1.0
        yi = lax.bitcast_convert_type(
            magic - lax.shift_right_arithmetic(
                lax.bitcast_convert_type(deg, jnp.int32), 1), jnp.float32)
        yi = yi * (1.5 - 0.5 * deg * yi * yi)
        yi = yi * (1.5 - 0.5 * deg * yi * yi)
        yi = yi * (1.5 - 0.5 * deg * yi * yi)
        g0 = yi * x0v[i]
        gv[i] = g0
        u9v[i] = 0.9 * (yi * yi)
        v0v[i] = 0.1 * g0
        dinvv[i] = yi

    pltpu.sync_copy(zv, acc_s.at[pl.ds(base, NPT)])
    pltpu.sync_copy(gv, g_hbm.at[pl.ds(base, NPT)])
    plsc.subcore_barrier()

    def edge_pass():
        for s in range(GA):
            pltpu.async_copy(g_hbm.at[idx_s.at[s]], gbuf.at[s], gsem.at[s])

        @pl.loop(0, CH)
        def _(cc):
            nxt = cc + GA
            s_n = lax.rem(nxt, NS)

            @pl.when(nxt < CH)
            def _():
                @pl.when(nxt >= NS)
                def _():
                    pltpu.make_async_copy(gbuf.at[s_n],
                                          acc_s.at[idx_d.at[nxt - NS]],
                                          ssem.at[s_n]).wait()

                pltpu.async_copy(g_hbm.at[idx_s.at[nxt]], gbuf.at[s_n],
                                 gsem.at[s_n])

            s = lax.rem(cc, NS)
            pltpu.make_async_copy(g_hbm.at[idx_s.at[cc]], gbuf.at[s],
                                  gsem.at[s]).wait()
            pltpu.async_copy(gbuf.at[s], acc_s.at[idx_d.at[cc]], ssem.at[s],
                             add=True)

        for s in range(NS):
            pltpu.make_async_copy(gbuf.at[s], acc_s.at[idx_d.at[0]],
                                  ssem.at[s]).wait()

    @pl.loop(0, K_PROP - 1)
    def _(k):
        edge_pass()
        plsc.subcore_barrier()
        pltpu.sync_copy(acc_s.at[pl.ds(base, NPT)], accv)
        pltpu.sync_copy(zv, acc_s.at[pl.ds(base, NPT)])

        @pl.loop(0, NPT)
        def _(i):
            gv[i] = u9v[i] * (accv[i] + gv[i]) + v0v[i]

        pltpu.sync_copy(gv, g_hbm.at[pl.ds(base, NPT)])
        plsc.subcore_barrier()

    edge_pass()
    plsc.subcore_barrier()
    pltpu.sync_copy(acc_s.at[pl.ds(base, NPT)], accv)

    @pl.loop(0, NPT)
    def _(i):
        h = 0.9 * dinvv[i] * (accv[i] + gv[i]) + 0.1 * x0v[i]
        tot = float(C) + jnp.sum(h)
        accv[i] = (1.0 + h) / tot

    pltpu.sync_copy(accv, soft_hbm.at[pl.ds(base, NPT)])


def kernel(x, edge_index, y, W1, b1, W2, b2, flow_z0, flow_alpha_raw,
           flow_beta_raw):
    ypad = jnp.concatenate(
        [y.astype(jnp.int32), jnp.full((80 * 128 - N,), C, jnp.int32)]
    ).reshape(80, 128)

    BLK = 1024
    NENC = 10 * BLK
    xpad = jnp.concatenate([x, jnp.zeros((NENC - N, 256), jnp.float32)], 0)
    z0T = flow_z0.reshape(C * RL, DL).T
    z0T = jnp.concatenate([z0T, jnp.zeros((DL, 64 - C * RL), jnp.float32)], 1)
    ar = jnp.concatenate([flow_alpha_raw.reshape(1, C * RL),
                          jnp.zeros((1, 64 - C * RL), jnp.float32)], 1)
    br = jnp.concatenate([flow_beta_raw.reshape(1, C * RL),
                          jnp.zeros((1, 64 - C * RL), jnp.float32)], 1)
    x0T = pl.pallas_call(
        _encoder_body,
        grid=(NENC // BLK,),
        in_specs=[
            pl.BlockSpec((80, 128), lambda i: (0, 0)),
            pl.BlockSpec((BLK, 256), lambda i: (i, 0)),
            pl.BlockSpec((256, 64), lambda i: (0, 0)),
            pl.BlockSpec((1, 64), lambda i: (0, 0)),
            pl.BlockSpec((DL, 64), lambda i: (0, 0)),
            pl.BlockSpec((DL, 1), lambda i: (0, 0)),
            pl.BlockSpec((DL, 64), lambda i: (0, 0)),
            pl.BlockSpec((1, 64), lambda i: (0, 0)),
            pl.BlockSpec((1, 64), lambda i: (0, 0)),
        ],
        out_specs=pl.BlockSpec((DL, BLK), lambda i: (0, i)),
        out_shape=jax.ShapeDtypeStruct((DL, NENC), jnp.float32),
        scratch_shapes=[pltpu.VMEM((1, 128), jnp.float32)],
    )(ypad, xpad, W1, b1.reshape(1, 64), W2.T, b2.reshape(DL, 1), z0T, ar, br)
    x0 = x0T.T[:N]

    src = edge_index[0].astype(jnp.int32)
    dst = edge_index[1].astype(jnp.int32)
    pad = NT * EPT - E
    src_pad = jnp.concatenate([src, jnp.zeros((pad,), jnp.int32)]
                              ).reshape(NT, CH, CW)
    dst_pad = jnp.concatenate([dst, jnp.full((pad,), N, jnp.int32)]
                              ).reshape(NT, CH, CW)

    soft16, _ = _make_appnp_sc()(src_pad, dst_pad, x0)
    return soft16[:, :C]
